# baseline (speedup 1.0000x reference)
import functools

import jax
import jax.numpy as jnp
from jax import lax
from jax.experimental import pallas as pl
from jax.experimental.pallas import tpu as pltpu
from jax.experimental.pallas import tpu_sc as plsc

BATCH = 16384
D = 64
NC = 2
NS = 16
L = 16
NW = NC * NS
BPW = BATCH // NW
HALF = BPW // 2
CHUNK = 128
NCHUNK = HALF // CHUNK


def _sc_body(hrow_u, hrow_g, off_u, off_g, utab, gtab, out,
             idx_u, idx_g, offb_u, offb_g, rows_u, rows_g, out_v, sem):
    wid = lax.axis_index("s") * NC + lax.axis_index("c")
    base = wid * BPW

    pltpu.sync_copy(off_u.at[pl.ds(base, BPW)], offb_u)
    pltpu.sync_copy(off_g.at[pl.ds(base, BPW)], offb_g)

    lanes = lax.iota(jnp.int32, L)
    ones = jnp.ones((L,), jnp.int32)

    for half in range(2):
        hbase = base + half * HALF
        for c in range(NCHUNK):
            pltpu.sync_copy(hrow_u.at[pl.ds(hbase + c * CHUNK, CHUNK)],
                            idx_u.at[c])
            pltpu.sync_copy(hrow_g.at[pl.ds(hbase + c * CHUNK, CHUNK)],
                            idx_g.at[c])
        for c in range(NCHUNK):
            pltpu.async_copy(utab.at[idx_u.at[c]],
                             rows_u.at[pl.ds(c * CHUNK, CHUNK)], sem)
            pltpu.async_copy(gtab.at[idx_g.at[c]],
                             rows_g.at[pl.ds(c * CHUNK, CHUNK)], sem)
        pltpu.make_async_copy(utab.at[pl.ds(0, HALF)], rows_u, sem).wait()
        pltpu.make_async_copy(gtab.at[pl.ds(0, HALF)], rows_g, sem).wait()

        def group(g, carry):
            rbase = g * L
            row_vec = rbase + lanes
            ucol = offb_u[pl.ds(half * HALF + rbase, L)]
            gcol = offb_g[pl.ds(half * HALF + rbase, L)]
            u = plsc.load_gather(rows_u, [row_vec, ucol])
            v = plsc.load_gather(rows_g, [row_vec, gcol])
            tot = u * v
            for _ in range(1, D):
                ucol = ucol + ones
                gcol = gcol + ones
                u = plsc.load_gather(rows_u, [row_vec, ucol])
                v = plsc.load_gather(rows_g, [row_vec, gcol])
                tot = tot + u * v
            sig = 1.0 / (1.0 + jnp.exp(-tot))
            out_v[pl.ds(half * HALF + rbase, L)] = sig
            return carry

        lax.fori_loop(0, HALF // L, group, 0)

    pltpu.sync_copy(out_v, out.at[pl.ds(base, BPW)])


@jax.jit
def _two_tower(hrow_u, hrow_g, off_u, off_g, user_tab2, game_tab2):
    mesh = plsc.VectorSubcoreMesh(core_axis_name="c", subcore_axis_name="s")
    f = pl.kernel(
        _sc_body,
        out_type=jax.ShapeDtypeStruct((BATCH,), jnp.float32),
        mesh=mesh,
        scratch_types=[
            pltpu.VMEM((NCHUNK, CHUNK), jnp.int32),
            pltpu.VMEM((NCHUNK, CHUNK), jnp.int32),
            pltpu.VMEM((BPW,), jnp.int32),
            pltpu.VMEM((BPW,), jnp.int32),
            pltpu.VMEM((HALF, 128), jnp.float32),
            pltpu.VMEM((HALF, 128), jnp.float32),
            pltpu.VMEM((BPW,), jnp.float32),
            pltpu.SemaphoreType.DMA,
        ],
        compiler_params=pltpu.CompilerParams(needs_layout_passes=False),
    )
    return f(hrow_u, hrow_g, off_u, off_g, user_tab2, game_tab2)


def kernel(user_ids, game_ids, user_table, game_table):
    user_ids = user_ids.astype(jnp.int32)
    game_ids = game_ids.astype(jnp.int32)
    u2 = user_table.reshape(user_table.shape[0] // 2, 2 * D)
    g2 = game_table.reshape(game_table.shape[0] // 2, 2 * D)
    return _two_tower(user_ids >> 1, game_ids >> 1,
                      (user_ids & 1) * D, (game_ids & 1) * D, u2, g2)


# speedup vs baseline: 1.1611x; 1.1611x over previous
import functools

import jax
import jax.numpy as jnp
from jax import lax
from jax.experimental import pallas as pl
from jax.experimental.pallas import tpu as pltpu
from jax.experimental.pallas import tpu_sc as plsc

BATCH = 16384
D = 64
PADW = 128
NC = 2
NS = 16
L = 16
NW = NC * NS
BPW = BATCH // NW
HALF = BPW // 2
CHUNK = 128
NCHUNK = HALF // CHUNK


def _sc_body(uids, gids, utab, gtab, out,
             idx_u, idx_g, rows_u, rows_g, tbuf, out_v, sem):
    wid = lax.axis_index("s") * NC + lax.axis_index("c")
    base = wid * BPW

    lanes16 = lax.iota(jnp.int32, L) * L

    for half in range(2):
        hbase = base + half * HALF
        for c in range(NCHUNK):
            pltpu.sync_copy(uids.at[pl.ds(hbase + c * CHUNK, CHUNK)],
                            idx_u.at[c])
            pltpu.sync_copy(gids.at[pl.ds(hbase + c * CHUNK, CHUNK)],
                            idx_g.at[c])
        for c in range(NCHUNK):
            pltpu.async_copy(utab.at[idx_u.at[c]],
                             rows_u.at[pl.ds(c * CHUNK, CHUNK)], sem)
            pltpu.async_copy(gtab.at[idx_g.at[c]],
                             rows_g.at[pl.ds(c * CHUNK, CHUNK)], sem)
        pltpu.make_async_copy(utab.at[pl.ds(0, HALF)], rows_u, sem).wait()
        pltpu.make_async_copy(gtab.at[pl.ds(0, HALF)], rows_g, sem).wait()

        def group(g, carry):
            rbase = g * L
            for r in range(L):
                row = rbase + r
                acc = (rows_u[row, pl.ds(0, L)] * rows_g[row, pl.ds(0, L)])
                for c in range(1, D // L):
                    acc = acc + (rows_u[row, pl.ds(c * L, L)] *
                                 rows_g[row, pl.ds(c * L, L)])
                tbuf[pl.ds(r * L, L)] = acc
            tot = plsc.load_gather(tbuf, [lanes16])
            for c in range(1, L):
                tot = tot + plsc.load_gather(tbuf, [lanes16 + c])
            sig = 1.0 / (1.0 + jnp.exp(-tot))
            out_v[pl.ds(half * HALF + rbase, L)] = sig
            return carry

        lax.fori_loop(0, HALF // L, group, 0)

    pltpu.sync_copy(out_v, out.at[pl.ds(base, BPW)])


@jax.jit
def _two_tower(user_ids, game_ids, user_pad, game_pad):
    mesh = plsc.VectorSubcoreMesh(core_axis_name="c", subcore_axis_name="s")
    f = pl.kernel(
        _sc_body,
        out_type=jax.ShapeDtypeStruct((BATCH,), jnp.float32),
        mesh=mesh,
        scratch_types=[
            pltpu.VMEM((NCHUNK, CHUNK), jnp.int32),
            pltpu.VMEM((NCHUNK, CHUNK), jnp.int32),
            pltpu.VMEM((HALF, PADW), jnp.float32),
            pltpu.VMEM((HALF, PADW), jnp.float32),
            pltpu.VMEM((L * L,), jnp.float32),
            pltpu.VMEM((BPW,), jnp.float32),
            pltpu.SemaphoreType.DMA,
        ],
        compiler_params=pltpu.CompilerParams(needs_layout_passes=False),
    )
    return f(user_ids, game_ids, user_pad, game_pad)


def kernel(user_ids, game_ids, user_table, game_table):
    user_ids = user_ids.astype(jnp.int32)
    game_ids = game_ids.astype(jnp.int32)
    up = jnp.pad(user_table, ((0, 0), (0, PADW - D)))
    gp = jnp.pad(game_table, ((0, 0), (0, PADW - D)))
    return _two_tower(user_ids, game_ids, up, gp)


# speedup vs baseline: 2.3347x; 2.0108x over previous
import functools

import jax
import jax.numpy as jnp
from jax import lax
from jax.experimental import pallas as pl
from jax.experimental.pallas import tpu as pltpu
from jax.experimental.pallas import tpu_sc as plsc

BATCH = 16384
D = 64
PADW = 128
NC = 2
NS = 16
L = 16
NW = NC * NS
BPW = BATCH // NW
HALF = BPW // 2
NU = 1000000
NG = 100000
ITU = (NU + 127) // 128
ITG = (NG + 127) // 128
CPW_U = ITU // NW
CPW_G = ITG // NW
FULL_U = ITU - 1 - 31 * CPW_U
FULL_G = ITG - 1 - 31 * CPW_G
PW_U = NU - (ITU - 1) * 128
PW_G = NG - (ITG - 1) * 128
NEMB = BATCH + L


def _splat(i):
    return jnp.zeros((L,), jnp.int32) + i


def _rd(ref, i):
    return plsc.load_gather(ref, [_splat(i)])[0]


def _sweep(tab, tailref, emb, sidb, pbb, stb, slots, rowbuf, semA, semB,
           semsc, wid, cpw, full31, itlast):
    base_it = wid * cpw
    nfull = lax.select(wid == NW - 1, jnp.int32(full31), jnp.int32(cpw))
    my_end = _rd(stb, wid + 1)
    p0 = _rd(stb, wid)

    jvecs = [jnp.arange(c * L, (c + 1) * L, dtype=jnp.int32)
             for c in range(D // L)]
    iota = lax.iota(jnp.int32, L)
    dump = BATCH + iota

    def fire(k, parity, sem):
        @pl.when(k < nfull)
        def _():
            col = pl.multiple_of((base_it + k) * 128, 128)
            pltpu.async_copy(tab.at[:, pl.ds(col, 128)],
                             slots.at[parity], sem)

    def drain_chunk(sem):
        pltpu.make_async_copy(tab.at[:, pl.ds(0, 128)],
                              slots.at[0], sem).wait()

    fire(jnp.int32(0), 0, semA)

    def extract(carry, git, slot_idx):
        def cond(c):
            p, r, pbv, nf = c
            return jnp.logical_and(p < my_end, _rd(sidb, p) >> 7 == git)

        def body(c):
            p, r, pbv, nf = c
            sid = _rd(sidb, p)
            lane = lax.bitwise_and(sid, jnp.int32(127))
            row = lax.rem(nf, 2) * L + r
            for cc in range(D // L):
                vals = plsc.load_gather(
                    slots, [_splat(slot_idx), jvecs[cc], _splat(lane)])
                rowbuf[row, pl.ds(cc * L, L)] = vals
            pbv = jnp.where(iota == r, _splat(_rd(pbb, p)), pbv)

            def flush(args):
                pbv_, nf_ = args
                @pl.when(nf_ > 0)
                def _():
                    pltpu.make_async_copy(
                        emb.at[pl.ds(0, L)], rowbuf.at[pl.ds(0, L)],
                        semsc).wait()
                half = pl.multiple_of(lax.rem(nf_, 2) * L, 8)
                pltpu.async_copy(rowbuf.at[pl.ds(half, L)],
                                 emb.at[pbv_], semsc)
                return dump, nf_ + 1

            pbv, nf = lax.cond(r == L - 1, flush,
                               lambda a: (a[0], a[1]), (pbv, nf))
            r = lax.rem(r + 1, L)
            return p + 1, r, pbv, nf

        return lax.while_loop(cond, body, carry)

    def pair(j, carry):
        k0 = j * 2

        def live0(c):
            fire(k0 + 1, 1, semB)
            drain_chunk(semA)
            return extract(c, base_it + k0, jnp.int32(0))

        carry = lax.cond(k0 < nfull, live0, lambda c: c, carry)

        def live1(c):
            fire(k0 + 2, 0, semA)
            drain_chunk(semB)
            return extract(c, base_it + k0 + 1, jnp.int32(1))

        return lax.cond(k0 + 1 < nfull, live1, lambda c: c, carry)

    carry = (p0, jnp.int32(0), dump, jnp.int32(0))
    carry = lax.fori_loop(0, (full31 + 1) // 2, pair, carry)

    def tail(c):
        pltpu.sync_copy(tailref, slots.at[0])
        return extract(c, jnp.int32(itlast), jnp.int32(0))

    carry = lax.cond(wid == NW - 1, tail, lambda c: c, carry)

    p, r, pbv, nf = carry

    def final_flush(args):
        pbv_, nf_ = args
        @pl.when(nf_ > 0)
        def _():
            pltpu.make_async_copy(emb.at[pl.ds(0, L)],
                                  rowbuf.at[pl.ds(0, L)], semsc).wait()
        half = pl.multiple_of(lax.rem(nf_, 2) * L, 8)
        pltpu.async_copy(rowbuf.at[pl.ds(half, L)], emb.at[pbv_], semsc)
        return pbv_, nf_ + 1

    pbv, nf = lax.cond(r > 0, final_flush, lambda a: (a[0], a[1]), (pbv, nf))

    @pl.when(nf > 0)
    def _():
        pltpu.make_async_copy(emb.at[pl.ds(0, L)],
                              rowbuf.at[pl.ds(0, L)], semsc).wait()


def _k1_body(sid_u, pb_u, st_u, sid_g, pb_g, st_g, utab, gtab, utail, gtail,
             uemb, gemb, sidb, pbb, stb, slots, rowbuf, semA, semB, semsc):
    wid = lax.axis_index("s") * NC + lax.axis_index("c")

    pltpu.sync_copy(sid_u, sidb)
    pltpu.sync_copy(pb_u, pbb)
    pltpu.sync_copy(st_u, stb)
    _sweep(utab, utail, uemb, sidb, pbb, stb, slots, rowbuf, semA, semB,
           semsc, wid, CPW_U, FULL_U, ITU - 1)

    pltpu.sync_copy(sid_g, sidb)
    pltpu.sync_copy(pb_g, pbb)
    pltpu.sync_copy(st_g, stb)
    _sweep(gtab, gtail, gemb, sidb, pbb, stb, slots, rowbuf, semA, semB,
           semsc, wid, CPW_G, FULL_G, ITG - 1)


def _k2_body(uemb, gemb, out, rows_u, rows_g, tbuf, out_v, sem):
    wid = lax.axis_index("s") * NC + lax.axis_index("c")
    base = wid * BPW
    lanes16 = lax.iota(jnp.int32, L) * L

    for half in range(2):
        hbase = base + half * HALF
        pltpu.sync_copy(uemb.at[pl.ds(hbase, HALF)], rows_u)
        pltpu.sync_copy(gemb.at[pl.ds(hbase, HALF)], rows_g)

        def group(g, carry):
            rbase = g * L
            for r in range(L):
                row = rbase + r
                acc = rows_u[row, pl.ds(0, L)] * rows_g[row, pl.ds(0, L)]
                for c in range(1, D // L):
                    acc = acc + (rows_u[row, pl.ds(c * L, L)] *
                                 rows_g[row, pl.ds(c * L, L)])
                tbuf[pl.ds(r * L, L)] = acc
            tot = plsc.load_gather(tbuf, [lanes16])
            for c in range(1, L):
                tot = tot + plsc.load_gather(tbuf, [lanes16 + c])
            sig = 1.0 / (1.0 + jnp.exp(-tot))
            out_v[pl.ds(half * HALF + rbase, L)] = sig
            return carry

        lax.fori_loop(0, HALF // L, group, 0)

    pltpu.sync_copy(out_v, out.at[pl.ds(base, BPW)])


@jax.jit
def _two_tower(user_ids, game_ids, user_t, game_t):
    iota = lax.iota(jnp.int32, BATCH)
    sid_u, pb_u = lax.sort_key_val(user_ids, iota)
    sid_g, pb_g = lax.sort_key_val(game_ids, iota)
    bnd_u = jnp.concatenate([jnp.arange(NW, dtype=jnp.int32) * (CPW_U * 128),
                             jnp.array([NU], jnp.int32)])
    bnd_g = jnp.concatenate([jnp.arange(NW, dtype=jnp.int32) * (CPW_G * 128),
                             jnp.array([NG], jnp.int32)])
    st_u = jnp.searchsorted(sid_u, bnd_u).astype(jnp.int32)
    st_g = jnp.searchsorted(sid_g, bnd_g).astype(jnp.int32)
    st_u = jnp.concatenate([st_u, jnp.zeros((7,), jnp.int32)])
    st_g = jnp.concatenate([st_g, jnp.zeros((7,), jnp.int32)])

    mesh = plsc.VectorSubcoreMesh(core_axis_name="c", subcore_axis_name="s")
    k1 = pl.kernel(
        _k1_body,
        out_type=(jax.ShapeDtypeStruct((NEMB, PADW), jnp.float32),
                  jax.ShapeDtypeStruct((NEMB, PADW), jnp.float32)),
        mesh=mesh,
        scratch_types=[
            pltpu.VMEM((BATCH,), jnp.int32),
            pltpu.VMEM((BATCH,), jnp.int32),
            pltpu.VMEM((NW + 8,), jnp.int32),
            pltpu.VMEM((2, D, 128), jnp.float32),
            pltpu.VMEM((2 * L, PADW), jnp.float32),
            pltpu.SemaphoreType.DMA,
            pltpu.SemaphoreType.DMA,
            pltpu.SemaphoreType.DMA,
        ],
        compiler_params=pltpu.CompilerParams(needs_layout_passes=False),
    )
    utail = jnp.pad(user_t[:, (ITU - 1) * 128:],
                    ((0, 0), (0, PADW - PW_U)))
    gtail = jnp.pad(game_t[:, (ITG - 1) * 128:],
                    ((0, 0), (0, PADW - PW_G)))
    uemb, gemb = k1(sid_u, pb_u, st_u, sid_g, pb_g, st_g, user_t, game_t,
                    utail, gtail)

    k2 = pl.kernel(
        _k2_body,
        out_type=jax.ShapeDtypeStruct((BATCH,), jnp.float32),
        mesh=mesh,
        scratch_types=[
            pltpu.VMEM((HALF, PADW), jnp.float32),
            pltpu.VMEM((HALF, PADW), jnp.float32),
            pltpu.VMEM((L * L,), jnp.float32),
            pltpu.VMEM((BPW,), jnp.float32),
            pltpu.SemaphoreType.DMA,
        ],
        compiler_params=pltpu.CompilerParams(needs_layout_passes=False),
    )
    return k2(uemb, gemb)


def kernel(user_ids, game_ids, user_table, game_table):
    user_ids = user_ids.astype(jnp.int32)
    game_ids = game_ids.astype(jnp.int32)
    return _two_tower(user_ids, game_ids, user_table.T, game_table.T)


# speedup vs baseline: 2.6933x; 1.1536x over previous
import functools

import jax
import jax.numpy as jnp
from jax import lax
from jax.experimental import pallas as pl
from jax.experimental.pallas import tpu as pltpu
from jax.experimental.pallas import tpu_sc as plsc

BATCH = 16384
D = 64
PADW = 128
NC = 2
NS = 16
L = 16
NW = NC * NS
BPW = BATCH // NW
HALF = BPW // 2
NU = 1000000
NG = 100000
ITU = (NU + 127) // 128
ITG = (NG + 127) // 128
IPC = 4
SLOTW = IPC * 128
CPW_U = ITU // NW
CPW_G = ITG // NW
CPWC_U = CPW_U // IPC
CPWC_G = CPW_G // IPC
FULLC_U = 62
FULLC_G = 9
TAIL_LO_U = 7812 * 128
TAIL_LO_G = 780 * 128
NEMB = BATCH + L


def _splat(i):
    return jnp.zeros((L,), jnp.int32) + i


def _rd(ref, i):
    return plsc.load_gather(ref, [_splat(i)])[0]


def _sweep(tab, tailref, emb, sidb, pbb, stb, slots, rowbuf, semA, semB,
           semsc, wid, cpwc, fullc31, tail_lo):
    base_c = wid * cpwc
    nfull = lax.select(wid == NW - 1, jnp.int32(fullc31), jnp.int32(cpwc))
    my_end = _rd(stb, wid + 1)
    p0 = _rd(stb, wid)

    jvecs = [jnp.arange(c * L, (c + 1) * L, dtype=jnp.int32)
             for c in range(D // L)]
    iota = lax.iota(jnp.int32, L)
    dump = BATCH + iota

    def fire(k, parity, sem):
        @pl.when(k < nfull)
        def _():
            col = pl.multiple_of((base_c + k) * SLOTW, 128)
            pltpu.async_copy(tab.at[:, pl.ds(col, SLOTW)],
                             slots.at[parity], sem)

    def drain_chunk(sem):
        pltpu.make_async_copy(tab.at[:, pl.ds(0, SLOTW)],
                              slots.at[0], sem).wait()

    fire(jnp.int32(0), 0, semA)

    def extract(carry, lo, slot_idx):
        hi = lo + SLOTW

        def cond(c):
            p, r, pbv, nf = c
            return jnp.logical_and(p < my_end, _rd(sidb, p) < hi)

        def body(c):
            p, r, pbv, nf = c
            sid = _rd(sidb, p)
            lane = sid - lo
            row = lax.rem(nf, 2) * L + r
            for cc in range(D // L):
                vals = plsc.load_gather(
                    slots, [_splat(slot_idx), jvecs[cc], _splat(lane)])
                rowbuf[row, pl.ds(cc * L, L)] = vals
            pbv = jnp.where(iota == r, _splat(_rd(pbb, p)), pbv)

            def flush(args):
                pbv_, nf_ = args
                @pl.when(nf_ > 0)
                def _():
                    pltpu.make_async_copy(
                        emb.at[pl.ds(0, L)], rowbuf.at[pl.ds(0, L)],
                        semsc).wait()
                half = pl.multiple_of(lax.rem(nf_, 2) * L, 8)
                pltpu.async_copy(rowbuf.at[pl.ds(half, L)],
                                 emb.at[pbv_], semsc)
                return dump, nf_ + 1

            pbv, nf = lax.cond(r == L - 1, flush,
                               lambda a: (a[0], a[1]), (pbv, nf))
            r = lax.rem(r + 1, L)
            return p + 1, r, pbv, nf

        return lax.while_loop(cond, body, carry)

    def pair(j, carry):
        k0 = j * 2

        def live0(c):
            fire(k0 + 1, 1, semB)
            drain_chunk(semA)
            return extract(c, (base_c + k0) * SLOTW, jnp.int32(0))

        carry = lax.cond(k0 < nfull, live0, lambda c: c, carry)

        def live1(c):
            fire(k0 + 2, 0, semA)
            drain_chunk(semB)
            return extract(c, (base_c + k0 + 1) * SLOTW, jnp.int32(1))

        return lax.cond(k0 + 1 < nfull, live1, lambda c: c, carry)

    carry = (p0, jnp.int32(0), dump, jnp.int32(0))
    carry = lax.fori_loop(0, (fullc31 + 1) // 2, pair, carry)

    def tail(c):
        pltpu.sync_copy(tailref, slots.at[0])
        return extract(c, jnp.int32(tail_lo), jnp.int32(0))

    carry = lax.cond(wid == NW - 1, tail, lambda c: c, carry)

    p, r, pbv, nf = carry

    def final_flush(args):
        pbv_, nf_ = args
        @pl.when(nf_ > 0)
        def _():
            pltpu.make_async_copy(emb.at[pl.ds(0, L)],
                                  rowbuf.at[pl.ds(0, L)], semsc).wait()
        half = pl.multiple_of(lax.rem(nf_, 2) * L, 8)
        pltpu.async_copy(rowbuf.at[pl.ds(half, L)], emb.at[pbv_], semsc)
        return pbv_, nf_ + 1

    pbv, nf = lax.cond(r > 0, final_flush, lambda a: (a[0], a[1]), (pbv, nf))

    @pl.when(nf > 0)
    def _():
        pltpu.make_async_copy(emb.at[pl.ds(0, L)],
                              rowbuf.at[pl.ds(0, L)], semsc).wait()


def _k1_body(sid_u, pb_u, st_u, sid_g, pb_g, st_g, utab, gtab, utail, gtail,
             uemb, gemb, sidb, pbb, stb, slots, rowbuf, semA, semB, semsc):
    wid = lax.axis_index("s") * NC + lax.axis_index("c")

    pltpu.sync_copy(sid_u, sidb)
    pltpu.sync_copy(pb_u, pbb)
    pltpu.sync_copy(st_u, stb)
    _sweep(utab, utail, uemb, sidb, pbb, stb, slots, rowbuf, semA, semB,
           semsc, wid, CPWC_U, FULLC_U, TAIL_LO_U)

    pltpu.sync_copy(sid_g, sidb)
    pltpu.sync_copy(pb_g, pbb)
    pltpu.sync_copy(st_g, stb)
    _sweep(gtab, gtail, gemb, sidb, pbb, stb, slots, rowbuf, semA, semB,
           semsc, wid, CPWC_G, FULLC_G, TAIL_LO_G)


def _k2_body(uemb, gemb, out, rows_u, rows_g, tbuf, out_v, sem):
    wid = lax.axis_index("s") * NC + lax.axis_index("c")
    base = wid * BPW
    lanes16 = lax.iota(jnp.int32, L) * L

    for half in range(2):
        hbase = base + half * HALF
        pltpu.sync_copy(uemb.at[pl.ds(hbase, HALF)], rows_u)
        pltpu.sync_copy(gemb.at[pl.ds(hbase, HALF)], rows_g)

        def group(g, carry):
            rbase = g * L
            for r in range(L):
                row = rbase + r
                acc = rows_u[row, pl.ds(0, L)] * rows_g[row, pl.ds(0, L)]
                for c in range(1, D // L):
                    acc = acc + (rows_u[row, pl.ds(c * L, L)] *
                                 rows_g[row, pl.ds(c * L, L)])
                tbuf[pl.ds(r * L, L)] = acc
            tot = plsc.load_gather(tbuf, [lanes16])
            for c in range(1, L):
                tot = tot + plsc.load_gather(tbuf, [lanes16 + c])
            sig = 1.0 / (1.0 + jnp.exp(-tot))
            out_v[pl.ds(half * HALF + rbase, L)] = sig
            return carry

        lax.fori_loop(0, HALF // L, group, 0)

    pltpu.sync_copy(out_v, out.at[pl.ds(base, BPW)])


@jax.jit
def _two_tower(user_ids, game_ids, user_t, game_t):
    iota = lax.iota(jnp.int32, BATCH)
    sid_u, pb_u = lax.sort_key_val(user_ids, iota)
    sid_g, pb_g = lax.sort_key_val(game_ids, iota)
    bnd_u = jnp.concatenate([jnp.arange(NW, dtype=jnp.int32) * (CPW_U * 128),
                             jnp.array([NU], jnp.int32)])
    bnd_g = jnp.concatenate([jnp.arange(NW, dtype=jnp.int32) * (CPW_G * 128),
                             jnp.array([NG], jnp.int32)])
    st_u = jnp.searchsorted(sid_u, bnd_u).astype(jnp.int32)
    st_g = jnp.searchsorted(sid_g, bnd_g).astype(jnp.int32)
    st_u = jnp.concatenate([st_u, jnp.zeros((7,), jnp.int32)])
    st_g = jnp.concatenate([st_g, jnp.zeros((7,), jnp.int32)])

    mesh = plsc.VectorSubcoreMesh(core_axis_name="c", subcore_axis_name="s")
    k1 = pl.kernel(
        _k1_body,
        out_type=(jax.ShapeDtypeStruct((NEMB, PADW), jnp.float32),
                  jax.ShapeDtypeStruct((NEMB, PADW), jnp.float32)),
        mesh=mesh,
        scratch_types=[
            pltpu.VMEM((BATCH,), jnp.int32),
            pltpu.VMEM((BATCH,), jnp.int32),
            pltpu.VMEM((NW + 8,), jnp.int32),
            pltpu.VMEM((2, D, SLOTW), jnp.float32),
            pltpu.VMEM((2 * L, PADW), jnp.float32),
            pltpu.SemaphoreType.DMA,
            pltpu.SemaphoreType.DMA,
            pltpu.SemaphoreType.DMA,
        ],
        compiler_params=pltpu.CompilerParams(needs_layout_passes=False),
    )
    utail = jnp.pad(user_t[:, TAIL_LO_U:],
                    ((0, 0), (0, SLOTW - (NU - TAIL_LO_U))))
    gtail = jnp.pad(game_t[:, TAIL_LO_G:],
                    ((0, 0), (0, SLOTW - (NG - TAIL_LO_G))))
    uemb, gemb = k1(sid_u, pb_u, st_u, sid_g, pb_g, st_g, user_t, game_t,
                    utail, gtail)

    k2 = pl.kernel(
        _k2_body,
        out_type=jax.ShapeDtypeStruct((BATCH,), jnp.float32),
        mesh=mesh,
        scratch_types=[
            pltpu.VMEM((HALF, PADW), jnp.float32),
            pltpu.VMEM((HALF, PADW), jnp.float32),
            pltpu.VMEM((L * L,), jnp.float32),
            pltpu.VMEM((BPW,), jnp.float32),
            pltpu.SemaphoreType.DMA,
        ],
        compiler_params=pltpu.CompilerParams(needs_layout_passes=False),
    )
    return k2(uemb, gemb)


def kernel(user_ids, game_ids, user_table, game_table):
    user_ids = user_ids.astype(jnp.int32)
    game_ids = game_ids.astype(jnp.int32)
    return _two_tower(user_ids, game_ids, user_table.T, game_table.T)


# speedup vs baseline: 2.8436x; 1.0558x over previous
import functools

import jax
import jax.numpy as jnp
from jax import lax
from jax.experimental import pallas as pl
from jax.experimental.pallas import tpu as pltpu
from jax.experimental.pallas import tpu_sc as plsc

BATCH = 16384
D = 64
PADW = 128
NC = 2
NS = 16
L = 16
NW = NC * NS
BPW = BATCH // NW
HALF = BPW // 2
NU = 1000000
NG = 100000
ITU = (NU + 127) // 128
ITG = (NG + 127) // 128
IPC = 4
SLOTW = IPC * 128
CPW_U = ITU // NW
CPW_G = ITG // NW
CPWC_U = CPW_U // IPC
CPWC_G = CPW_G // IPC
FULLC_U = 62
FULLC_G = 9
TAIL_LO_U = 7812 * 128
TAIL_LO_G = 780 * 128
NEMB = BATCH + L


def _splat(i):
    return jnp.zeros((L,), jnp.int32) + i


def _rd(ref, i):
    return plsc.load_gather(ref, [_splat(i)])[0]


def _sweep(tab, tailref, emb, sidb, pbb, stb, slots, rowbuf, semA, semB,
           semsc, wid, cpwc, fullc31, tail_lo):
    base_c = wid * cpwc
    nfull = lax.select(wid == NW - 1, jnp.int32(fullc31), jnp.int32(cpwc))
    my_end = _rd(stb, wid + 1)
    p0 = _rd(stb, wid)

    jvecs = [jnp.arange(c * L, (c + 1) * L, dtype=jnp.int32)
             for c in range(D // L)]
    iota = lax.iota(jnp.int32, L)
    dump = BATCH + iota

    def fire(k, parity, sem):
        @pl.when(k < nfull)
        def _():
            col = pl.multiple_of((base_c + k) * SLOTW, 128)
            pltpu.async_copy(tab.at[:, pl.ds(col, SLOTW)],
                             slots.at[parity], sem)

    def drain_chunk(sem):
        pltpu.make_async_copy(tab.at[:, pl.ds(0, SLOTW)],
                              slots.at[0], sem).wait()

    fire(jnp.int32(0), 0, semA)

    def extract(carry, lo, slot_idx):
        hi = lo + SLOTW

        def cond(c):
            p, r, pbv, nf = c
            return jnp.logical_and(p < my_end, _rd(sidb, p) < hi)

        def body(c):
            p, r, pbv, nf = c
            sid = _rd(sidb, p)
            lane = sid - lo
            row = lax.rem(nf, 2) * L + r
            for cc in range(D // L):
                vals = plsc.load_gather(
                    slots, [_splat(slot_idx), jvecs[cc], _splat(lane)])
                rowbuf[row, pl.ds(cc * L, L)] = vals
            pbv = jnp.where(iota == r, _splat(_rd(pbb, p)), pbv)

            def flush(args):
                pbv_, nf_ = args
                @pl.when(nf_ > 0)
                def _():
                    pltpu.make_async_copy(
                        emb.at[pl.ds(0, L)], rowbuf.at[pl.ds(0, L)],
                        semsc).wait()
                half = pl.multiple_of(lax.rem(nf_, 2) * L, 8)
                pltpu.async_copy(rowbuf.at[pl.ds(half, L)],
                                 emb.at[pbv_], semsc)
                return dump, nf_ + 1

            pbv, nf = lax.cond(r == L - 1, flush,
                               lambda a: (a[0], a[1]), (pbv, nf))
            r = lax.rem(r + 1, L)
            return p + 1, r, pbv, nf

        return lax.while_loop(cond, body, carry)

    def pair(j, carry):
        k0 = j * 2

        def live0(c):
            fire(k0 + 1, 1, semB)
            drain_chunk(semA)
            return extract(c, (base_c + k0) * SLOTW, jnp.int32(0))

        carry = lax.cond(k0 < nfull, live0, lambda c: c, carry)

        def live1(c):
            fire(k0 + 2, 0, semA)
            drain_chunk(semB)
            return extract(c, (base_c + k0 + 1) * SLOTW, jnp.int32(1))

        return lax.cond(k0 + 1 < nfull, live1, lambda c: c, carry)

    carry = (p0, jnp.int32(0), dump, jnp.int32(0))
    carry = lax.fori_loop(0, (fullc31 + 1) // 2, pair, carry)

    def tail(c):
        pltpu.sync_copy(tailref, slots.at[0])
        return extract(c, jnp.int32(tail_lo), jnp.int32(0))

    carry = lax.cond(wid == NW - 1, tail, lambda c: c, carry)

    p, r, pbv, nf = carry

    def final_flush(args):
        pbv_, nf_ = args
        @pl.when(nf_ > 0)
        def _():
            pltpu.make_async_copy(emb.at[pl.ds(0, L)],
                                  rowbuf.at[pl.ds(0, L)], semsc).wait()
        half = pl.multiple_of(lax.rem(nf_, 2) * L, 8)
        pltpu.async_copy(rowbuf.at[pl.ds(half, L)], emb.at[pbv_], semsc)
        return pbv_, nf_ + 1

    pbv, nf = lax.cond(r > 0, final_flush, lambda a: (a[0], a[1]), (pbv, nf))

    @pl.when(nf > 0)
    def _():
        pltpu.make_async_copy(emb.at[pl.ds(0, L)],
                              rowbuf.at[pl.ds(0, L)], semsc).wait()


def _k1_body(sid_u, pb_u, st_u, sid_g, pb_g, st_g, utab, gtab, utail, gtail,
             uemb, gemb, sidb, pbb, stb, slots, rowbuf, semA, semB, semsc):
    wid = lax.axis_index("s") * NC + lax.axis_index("c")

    pltpu.sync_copy(sid_u, sidb)
    pltpu.sync_copy(pb_u, pbb)
    pltpu.sync_copy(st_u, stb)
    _sweep(utab, utail, uemb, sidb, pbb, stb, slots, rowbuf, semA, semB,
           semsc, wid, CPWC_U, FULLC_U, TAIL_LO_U)

    pltpu.sync_copy(sid_g, sidb)
    pltpu.sync_copy(pb_g, pbb)
    pltpu.sync_copy(st_g, stb)
    _sweep(gtab, gtail, gemb, sidb, pbb, stb, slots, rowbuf, semA, semB,
           semsc, wid, CPWC_G, FULLC_G, TAIL_LO_G)


def _k2_body(uemb, gemb, out, rows_u, rows_g, tbuf, out_v, sem):
    wid = lax.axis_index("s") * NC + lax.axis_index("c")
    base = wid * BPW
    lanes16 = lax.iota(jnp.int32, L) * L

    for half in range(2):
        hbase = base + half * HALF
        pltpu.sync_copy(uemb.at[pl.ds(hbase, HALF)], rows_u)
        pltpu.sync_copy(gemb.at[pl.ds(hbase, HALF)], rows_g)

        def group(g, carry):
            rbase = g * L
            for r in range(L):
                row = rbase + r
                acc = rows_u[row, pl.ds(0, L)] * rows_g[row, pl.ds(0, L)]
                for c in range(1, D // L):
                    acc = acc + (rows_u[row, pl.ds(c * L, L)] *
                                 rows_g[row, pl.ds(c * L, L)])
                tbuf[pl.ds(r * L, L)] = acc
            tot = plsc.load_gather(tbuf, [lanes16])
            for c in range(1, L):
                tot = tot + plsc.load_gather(tbuf, [lanes16 + c])
            sig = 1.0 / (1.0 + jnp.exp(-tot))
            out_v[pl.ds(half * HALF + rbase, L)] = sig
            return carry

        lax.fori_loop(0, HALF // L, group, 0)

    pltpu.sync_copy(out_v, out.at[pl.ds(base, BPW)])


@jax.jit
def _two_tower(user_ids, game_ids, user_t, game_t):
    iota = lax.iota(jnp.int32, BATCH)
    sid_u, pb_u = lax.sort_key_val(user_ids, iota)
    sid_g, pb_g = lax.sort_key_val(game_ids, iota)
    bnd_u = jnp.concatenate([jnp.arange(NW, dtype=jnp.int32) * (CPW_U * 128),
                             jnp.array([NU], jnp.int32)])
    bnd_g = jnp.concatenate([jnp.arange(NW, dtype=jnp.int32) * (CPW_G * 128),
                             jnp.array([NG], jnp.int32)])
    st_u = jnp.sum(sid_u[None, :] < bnd_u[:, None], axis=1,
                   dtype=jnp.int32)
    st_g = jnp.sum(sid_g[None, :] < bnd_g[:, None], axis=1,
                   dtype=jnp.int32)
    st_u = jnp.concatenate([st_u, jnp.zeros((7,), jnp.int32)])
    st_g = jnp.concatenate([st_g, jnp.zeros((7,), jnp.int32)])

    mesh = plsc.VectorSubcoreMesh(core_axis_name="c", subcore_axis_name="s")
    k1 = pl.kernel(
        _k1_body,
        out_type=(jax.ShapeDtypeStruct((NEMB, PADW), jnp.float32),
                  jax.ShapeDtypeStruct((NEMB, PADW), jnp.float32)),
        mesh=mesh,
        scratch_types=[
            pltpu.VMEM((BATCH,), jnp.int32),
            pltpu.VMEM((BATCH,), jnp.int32),
            pltpu.VMEM((NW + 8,), jnp.int32),
            pltpu.VMEM((2, D, SLOTW), jnp.float32),
            pltpu.VMEM((2 * L, PADW), jnp.float32),
            pltpu.SemaphoreType.DMA,
            pltpu.SemaphoreType.DMA,
            pltpu.SemaphoreType.DMA,
        ],
        compiler_params=pltpu.CompilerParams(needs_layout_passes=False),
    )
    utail = jnp.pad(user_t[:, TAIL_LO_U:],
                    ((0, 0), (0, SLOTW - (NU - TAIL_LO_U))))
    gtail = jnp.pad(game_t[:, TAIL_LO_G:],
                    ((0, 0), (0, SLOTW - (NG - TAIL_LO_G))))
    uemb, gemb = k1(sid_u, pb_u, st_u, sid_g, pb_g, st_g, user_t, game_t,
                    utail, gtail)

    k2 = pl.kernel(
        _k2_body,
        out_type=jax.ShapeDtypeStruct((BATCH,), jnp.float32),
        mesh=mesh,
        scratch_types=[
            pltpu.VMEM((HALF, PADW), jnp.float32),
            pltpu.VMEM((HALF, PADW), jnp.float32),
            pltpu.VMEM((L * L,), jnp.float32),
            pltpu.VMEM((BPW,), jnp.float32),
            pltpu.SemaphoreType.DMA,
        ],
        compiler_params=pltpu.CompilerParams(needs_layout_passes=False),
    )
    return k2(uemb, gemb)


def kernel(user_ids, game_ids, user_table, game_table):
    user_ids = user_ids.astype(jnp.int32)
    game_ids = game_ids.astype(jnp.int32)
    return _two_tower(user_ids, game_ids, user_table.T, game_table.T)


# speedup vs baseline: 2.9747x; 1.0461x over previous
import functools

import jax
import jax.numpy as jnp
from jax import lax
from jax.experimental import pallas as pl
from jax.experimental.pallas import tpu as pltpu
from jax.experimental.pallas import tpu_sc as plsc

BATCH = 16384
D = 64
PADW = 128
NC = 2
NS = 16
L = 16
NW = NC * NS
BPW = BATCH // NW
HALF = BPW // 2
NU = 1000000
NG = 100000
ITU = (NU + 127) // 128
ITG = (NG + 127) // 128
IPC = 2
SLOTW = IPC * 128
CPW_U = ITU // NW
CPW_G = ITG // NW
CPWC_U = CPW_U // IPC
CPWC_G = CPW_G // IPC
FULLC_U = 124
FULLC_G = 18
TAIL_LO_U = 7812 * 128
TAIL_LO_G = 780 * 128
NEMB = BATCH + L


def _splat(i):
    return jnp.zeros((L,), jnp.int32) + i


def _rd(ref, i):
    return plsc.load_gather(ref, [_splat(i)])[0]


def _sweep(tab, tailref, emb, sidb, pbb, stb, slots, rowbuf, semA, semB,
           semC, semsc, wid, cpwc, fullc31, tail_lo):
    base_c = wid * cpwc
    nfull = lax.select(wid == NW - 1, jnp.int32(fullc31), jnp.int32(cpwc))
    my_end = _rd(stb, wid + 1)
    p0 = _rd(stb, wid)

    jvecs = [jnp.arange(c * L, (c + 1) * L, dtype=jnp.int32)
             for c in range(D // L)]
    iota = lax.iota(jnp.int32, L)
    dump = BATCH + iota

    def fire(k, parity, sem):
        @pl.when(k < nfull)
        def _():
            col = pl.multiple_of((base_c + k) * SLOTW, 128)
            pltpu.async_copy(tab.at[:, pl.ds(col, SLOTW)],
                             slots.at[parity], sem)

    def drain_chunk(sem):
        pltpu.make_async_copy(tab.at[:, pl.ds(0, SLOTW)],
                              slots.at[0], sem).wait()

    fire(jnp.int32(0), 0, semA)
    fire(jnp.int32(1), 1, semB)

    def extract(carry, lo, slot_idx):
        hi = lo + SLOTW

        def cond(c):
            p, r, pbv, nf = c
            return jnp.logical_and(p < my_end, _rd(sidb, p) < hi)

        def body(c):
            p, r, pbv, nf = c
            sid = _rd(sidb, p)
            lane = sid - lo
            row = lax.rem(nf, 2) * L + r
            for cc in range(D // L):
                vals = plsc.load_gather(
                    slots, [_splat(slot_idx), jvecs[cc], _splat(lane)])
                rowbuf[row, pl.ds(cc * L, L)] = vals
            pbv = jnp.where(iota == r, _splat(_rd(pbb, p)), pbv)

            def flush(args):
                pbv_, nf_ = args
                @pl.when(nf_ > 0)
                def _():
                    pltpu.make_async_copy(
                        emb.at[pl.ds(0, L)], rowbuf.at[pl.ds(0, L)],
                        semsc).wait()
                half = pl.multiple_of(lax.rem(nf_, 2) * L, 8)
                pltpu.async_copy(rowbuf.at[pl.ds(half, L)],
                                 emb.at[pbv_], semsc)
                return dump, nf_ + 1

            pbv, nf = lax.cond(r == L - 1, flush,
                               lambda a: (a[0], a[1]), (pbv, nf))
            r = lax.rem(r + 1, L)
            return p + 1, r, pbv, nf

        return lax.while_loop(cond, body, carry)

    def triple(j, carry):
        k0 = j * 3
        for d, (par, sem) in enumerate([(0, semA), (1, semB), (2, semC)]):
            def live(c, k=k0 + d, par=par, sem=sem):
                fire(k + 2, (par + 2) % 3, [semA, semB, semC][(par + 2) % 3])
                drain_chunk(sem)
                return extract(c, (base_c + k) * SLOTW, jnp.int32(par))
            carry = lax.cond(k0 + d < nfull, live, lambda c: c, carry)
        return carry

    carry = (p0, jnp.int32(0), dump, jnp.int32(0))
    carry = lax.fori_loop(0, (fullc31 + 2) // 3, triple, carry)

    def tail(c):
        pltpu.sync_copy(tailref, slots.at[0])
        return extract(c, jnp.int32(tail_lo), jnp.int32(0))

    carry = lax.cond(wid == NW - 1, tail, lambda c: c, carry)

    p, r, pbv, nf = carry

    def final_flush(args):
        pbv_, nf_ = args
        @pl.when(nf_ > 0)
        def _():
            pltpu.make_async_copy(emb.at[pl.ds(0, L)],
                                  rowbuf.at[pl.ds(0, L)], semsc).wait()
        half = pl.multiple_of(lax.rem(nf_, 2) * L, 8)
        pltpu.async_copy(rowbuf.at[pl.ds(half, L)], emb.at[pbv_], semsc)
        return pbv_, nf_ + 1

    pbv, nf = lax.cond(r > 0, final_flush, lambda a: (a[0], a[1]), (pbv, nf))

    @pl.when(nf > 0)
    def _():
        pltpu.make_async_copy(emb.at[pl.ds(0, L)],
                              rowbuf.at[pl.ds(0, L)], semsc).wait()


def _k1_body(sid_u, pb_u, st_u, sid_g, pb_g, st_g, utab, gtab, utail, gtail,
             uemb, gemb, sidb, pbb, stb, slots, rowbuf, semA, semB, semC,
             semsc):
    wid = lax.axis_index("s") * NC + lax.axis_index("c")

    pltpu.sync_copy(sid_u, sidb)
    pltpu.sync_copy(pb_u, pbb)
    pltpu.sync_copy(st_u, stb)
    _sweep(utab, utail, uemb, sidb, pbb, stb, slots, rowbuf, semA, semB,
           semC, semsc, wid, CPWC_U, FULLC_U, TAIL_LO_U)

    pltpu.sync_copy(sid_g, sidb)
    pltpu.sync_copy(pb_g, pbb)
    pltpu.sync_copy(st_g, stb)
    _sweep(gtab, gtail, gemb, sidb, pbb, stb, slots, rowbuf, semA, semB,
           semC, semsc, wid, CPWC_G, FULLC_G, TAIL_LO_G)


def _k2_body(uemb, gemb, out, rows_u, rows_g, tbuf, out_v, sem):
    wid = lax.axis_index("s") * NC + lax.axis_index("c")
    base = wid * BPW
    lanes16 = lax.iota(jnp.int32, L) * L

    for half in range(2):
        hbase = base + half * HALF
        pltpu.sync_copy(uemb.at[pl.ds(hbase, HALF)], rows_u)
        pltpu.sync_copy(gemb.at[pl.ds(hbase, HALF)], rows_g)

        def group(g, carry):
            rbase = g * L
            for r in range(L):
                row = rbase + r
                acc = rows_u[row, pl.ds(0, L)] * rows_g[row, pl.ds(0, L)]
                for c in range(1, D // L):
                    acc = acc + (rows_u[row, pl.ds(c * L, L)] *
                                 rows_g[row, pl.ds(c * L, L)])
                tbuf[pl.ds(r * L, L)] = acc
            tot = plsc.load_gather(tbuf, [lanes16])
            for c in range(1, L):
                tot = tot + plsc.load_gather(tbuf, [lanes16 + c])
            sig = 1.0 / (1.0 + jnp.exp(-tot))
            out_v[pl.ds(half * HALF + rbase, L)] = sig
            return carry

        lax.fori_loop(0, HALF // L, group, 0)

    pltpu.sync_copy(out_v, out.at[pl.ds(base, BPW)])


@jax.jit
def _two_tower(user_ids, game_ids, user_t, game_t):
    iota = lax.iota(jnp.int32, BATCH)
    sid_u, pb_u = lax.sort_key_val(user_ids, iota)
    sid_g, pb_g = lax.sort_key_val(game_ids, iota)
    bnd_u = jnp.concatenate([jnp.arange(NW, dtype=jnp.int32) * (CPW_U * 128),
                             jnp.array([NU], jnp.int32)])
    bnd_g = jnp.concatenate([jnp.arange(NW, dtype=jnp.int32) * (CPW_G * 128),
                             jnp.array([NG], jnp.int32)])
    st_u = jnp.sum(sid_u[None, :] < bnd_u[:, None], axis=1,
                   dtype=jnp.int32)
    st_g = jnp.sum(sid_g[None, :] < bnd_g[:, None], axis=1,
                   dtype=jnp.int32)
    st_u = jnp.concatenate([st_u, jnp.zeros((7,), jnp.int32)])
    st_g = jnp.concatenate([st_g, jnp.zeros((7,), jnp.int32)])

    mesh = plsc.VectorSubcoreMesh(core_axis_name="c", subcore_axis_name="s")
    k1 = pl.kernel(
        _k1_body,
        out_type=(jax.ShapeDtypeStruct((NEMB, PADW), jnp.float32),
                  jax.ShapeDtypeStruct((NEMB, PADW), jnp.float32)),
        mesh=mesh,
        scratch_types=[
            pltpu.VMEM((BATCH,), jnp.int32),
            pltpu.VMEM((BATCH,), jnp.int32),
            pltpu.VMEM((NW + 8,), jnp.int32),
            pltpu.VMEM((3, D, SLOTW), jnp.float32),
            pltpu.VMEM((2 * L, PADW), jnp.float32),
            pltpu.SemaphoreType.DMA,
            pltpu.SemaphoreType.DMA,
            pltpu.SemaphoreType.DMA,
            pltpu.SemaphoreType.DMA,
        ],
        compiler_params=pltpu.CompilerParams(needs_layout_passes=False),
    )
    utail = jnp.pad(user_t[:, TAIL_LO_U:],
                    ((0, 0), (0, SLOTW - (NU - TAIL_LO_U))))
    gtail = jnp.pad(game_t[:, TAIL_LO_G:],
                    ((0, 0), (0, SLOTW - (NG - TAIL_LO_G))))
    uemb, gemb = k1(sid_u, pb_u, st_u, sid_g, pb_g, st_g, user_t, game_t,
                    utail, gtail)

    k2 = pl.kernel(
        _k2_body,
        out_type=jax.ShapeDtypeStruct((BATCH,), jnp.float32),
        mesh=mesh,
        scratch_types=[
            pltpu.VMEM((HALF, PADW), jnp.float32),
            pltpu.VMEM((HALF, PADW), jnp.float32),
            pltpu.VMEM((L * L,), jnp.float32),
            pltpu.VMEM((BPW,), jnp.float32),
            pltpu.SemaphoreType.DMA,
        ],
        compiler_params=pltpu.CompilerParams(needs_layout_passes=False),
    )
    return k2(uemb, gemb)


def kernel(user_ids, game_ids, user_table, game_table):
    user_ids = user_ids.astype(jnp.int32)
    game_ids = game_ids.astype(jnp.int32)
    return _two_tower(user_ids, game_ids, user_table.T, game_table.T)
